# baseline (speedup 1.0000x reference)
import math

import jax
import jax.numpy as jnp
from jax.experimental import pallas as pl
from jax.experimental.pallas import tpu as pltpu

_LN_EPS = 1e-12
_SQRT_HALF = 1.0 / math.sqrt(2.0)
_NC = 4


def _cdiv(a, b):
    return (a + b - 1) // b


def _transform_kernel(x_ref, w1_ref, b1_ref, gamma_ref, beta_ref, h_ref):
    x = x_ref[...].astype(jnp.bfloat16)
    w = w1_ref[...].astype(jnp.bfloat16)
    y = jax.lax.dot_general(x, w, (((1,), (1,)), ((), ())),
                            preferred_element_type=jnp.float32)
    y = y + b1_ref[...]
    y = y * 0.5 * (1.0 + jax.lax.erf(y * _SQRT_HALF))
    mean = jnp.mean(y, axis=-1, keepdims=True)
    centered = y - mean
    var = jnp.mean(centered * centered, axis=-1, keepdims=True)
    y = centered * jax.lax.rsqrt(var + _LN_EPS)
    y = y * gamma_ref[...] + beta_ref[...]
    h_ref[...] = y.astype(h_ref.dtype)


def _manual_decoder_kernel(h_ref, w_hbm, b_hbm, out_hbm,
                           wbuf, bbuf, acc, wsem, bsem, osem):
    Bb, Ss, Vv = out_hbm.shape
    tv = wbuf.shape[1]
    nj2 = pl.num_programs(1)
    c = pl.program_id(0)
    j2 = pl.program_id(1)
    jg = c * nj2 + j2
    ch = Bb // _NC
    slot = jax.lax.rem(j2, 2)
    nxt = 1 - slot

    def w_copy(s, t):
        return pltpu.make_async_copy(
            w_hbm.at[pl.ds(t * tv, tv), :], wbuf.at[s], wsem.at[s])

    def b_copy(s, t):
        return pltpu.make_async_copy(
            b_hbm.at[:, pl.ds(t * tv, tv)], bbuf.at[s], bsem.at[s])

    def o_copy(s, cb, t):
        return pltpu.make_async_copy(
            acc.at[s, pl.ds(cb * ch, ch)],
            out_hbm.at[pl.ds(cb * ch, ch), :, pl.ds(t * tv, tv)],
            osem.at[s, cb])

    @pl.when(j2 == 0)
    def _():
        w_copy(slot, jg).start()
        b_copy(slot, jg).start()

    w_copy(slot, jg).wait()
    b_copy(slot, jg).wait()

    @pl.when(j2 + 1 < nj2)
    def _():
        w_copy(nxt, jg + 1).start()
        b_copy(nxt, jg + 1).start()

    @pl.when(j2 >= 2)
    def _():
        for cb in range(_NC):
            o_copy(slot, cb, jg - 2).wait()

    wv = wbuf[slot].astype(jnp.bfloat16)
    logits = jax.lax.dot_general(h_ref[...], wv, (((1,), (1,)), ((), ())),
                                 preferred_element_type=jnp.float32)
    logits = (logits + bbuf[slot]).reshape(Bb, Ss, tv)

    @pl.when(slot == 0)
    def _():
        acc[0] = logits

    @pl.when(slot == 1)
    def _():
        acc[1] = logits

    for cb in range(_NC):
        o_copy(slot, cb, jg).start()

    @pl.when(j2 == nj2 - 1)
    def _():
        for cb in range(_NC):
            o_copy(nxt, cb, jg - 1).wait()
        for cb in range(_NC):
            o_copy(slot, cb, jg).wait()


def _simple_decoder_kernel(h_ref, wv_ref, bv_ref, out_ref):
    w = wv_ref[...].astype(jnp.bfloat16)
    logits = jax.lax.dot_general(h_ref[...], w, (((1,), (1,)), ((), ())),
                                 preferred_element_type=jnp.float32)
    logits = logits + bv_ref[...]
    out_ref[...] = logits.reshape(out_ref.shape).astype(out_ref.dtype)


def _tail_decoder_kernel(prev_ref, h_ref, wv_ref, bv_ref, out_ref):
    del prev_ref
    w = wv_ref[...].astype(jnp.bfloat16)
    logits = jax.lax.dot_general(h_ref[...], w, (((1,), (1,)), ((), ())),
                                 preferred_element_type=jnp.float32)
    logits = logits + bv_ref[...]
    out_ref[...] = logits.reshape(out_ref.shape).astype(out_ref.dtype)


def kernel(x, w1, b1, gamma, beta, dec_w, dec_b):
    B, S, H = x.shape
    V = dec_w.shape[0]
    M = B * S

    x2 = x.reshape(M, H)
    b1_2 = b1.reshape(1, H).astype(jnp.float32)
    gamma_2 = gamma.reshape(1, H).astype(jnp.float32)
    beta_2 = beta.reshape(1, H).astype(jnp.float32)
    dec_b_2 = dec_b.reshape(1, V).astype(jnp.float32)

    tm = min(512, M)
    h = pl.pallas_call(
        _transform_kernel,
        out_shape=jax.ShapeDtypeStruct((M, H), jnp.bfloat16),
        grid=(_cdiv(M, tm),),
        in_specs=[
            pl.BlockSpec((tm, H), lambda i: (i, 0)),
            pl.BlockSpec((H, H), lambda i: (0, 0)),
            pl.BlockSpec((1, H), lambda i: (0, 0)),
            pl.BlockSpec((1, H), lambda i: (0, 0)),
            pl.BlockSpec((1, H), lambda i: (0, 0)),
        ],
        out_specs=pl.BlockSpec((tm, H), lambda i: (i, 0)),
        compiler_params=pltpu.CompilerParams(
            dimension_semantics=("parallel",),
            vmem_limit_bytes=64 * 1024 * 1024,
        ),
        cost_estimate=pl.CostEstimate(
            flops=2 * M * H * H,
            transcendentals=M * H,
            bytes_accessed=4 * (M * H + H * H + 3 * H) + 2 * M * H,
        ),
    )(x2, w1, b1_2, gamma_2, beta_2)

    tv = 1024
    njt = _cdiv(V, tv)
    n_full = V // tv
    nja = (n_full // 2) * 2
    n_tail = njt - nja
    use_manual = (nja >= 4 and n_tail >= 1 and B % _NC == 0 and M == B * S)
    if use_manual:
        nj2 = nja // 2
        out_a = pl.pallas_call(
            _manual_decoder_kernel,
            out_shape=jax.ShapeDtypeStruct((B, S, V), jnp.float32),
            grid=(2, nj2),
            in_specs=[
                pl.BlockSpec((M, H), lambda c, j: (0, 0)),
                pl.BlockSpec(memory_space=pl.ANY),
                pl.BlockSpec(memory_space=pl.ANY),
            ],
            out_specs=pl.BlockSpec(memory_space=pl.ANY),
            scratch_shapes=[
                pltpu.VMEM((2, tv, H), jnp.float32),
                pltpu.VMEM((2, 1, tv), jnp.float32),
                pltpu.VMEM((2, B, S, tv), jnp.float32),
                pltpu.SemaphoreType.DMA((2,)),
                pltpu.SemaphoreType.DMA((2,)),
                pltpu.SemaphoreType.DMA((2, _NC)),
            ],
            compiler_params=pltpu.CompilerParams(
                dimension_semantics=("parallel", "arbitrary"),
                vmem_limit_bytes=60 * 1024 * 1024,
            ),
            cost_estimate=pl.CostEstimate(
                flops=2 * M * H * nja * tv,
                transcendentals=0,
                bytes_accessed=2 * M * H + 4 * (H * V + V + M * nja * tv),
            ),
        )(h, dec_w, dec_b_2)
        out = pl.pallas_call(
            _tail_decoder_kernel,
            out_shape=jax.ShapeDtypeStruct((B, S, V), jnp.float32),
            grid=(n_tail,),
            in_specs=[
                pl.BlockSpec(memory_space=pl.ANY),
                pl.BlockSpec((M, H), lambda j: (0, 0)),
                pl.BlockSpec((tv, H), lambda j: (nja + j, 0)),
                pl.BlockSpec((1, tv), lambda j: (0, nja + j)),
            ],
            out_specs=pl.BlockSpec((B, S, tv), lambda j: (0, 0, nja + j)),
            input_output_aliases={0: 0},
            compiler_params=pltpu.CompilerParams(
                dimension_semantics=("parallel",),
                vmem_limit_bytes=64 * 1024 * 1024,
            ),
            cost_estimate=pl.CostEstimate(
                flops=2 * M * H * n_tail * tv,
                transcendentals=0,
                bytes_accessed=2 * M * H + 4 * (H * n_tail * tv
                                                + n_tail * tv
                                                + M * n_tail * tv),
            ),
        )(out_a, h, dec_w, dec_b_2)
    else:
        out = pl.pallas_call(
            _simple_decoder_kernel,
            out_shape=jax.ShapeDtypeStruct((B, S, V), jnp.float32),
            grid=(njt,),
            in_specs=[
                pl.BlockSpec((M, H), lambda j: (0, 0)),
                pl.BlockSpec((tv, H), lambda j: (j, 0)),
                pl.BlockSpec((1, tv), lambda j: (0, j)),
            ],
            out_specs=pl.BlockSpec((B, S, tv), lambda j: (0, 0, j)),
            compiler_params=pltpu.CompilerParams(
                dimension_semantics=("parallel",),
                vmem_limit_bytes=64 * 1024 * 1024,
            ),
            cost_estimate=pl.CostEstimate(
                flops=2 * M * H * V,
                transcendentals=0,
                bytes_accessed=2 * M * H + 4 * (H * V + V + M * V),
            ),
        )(h, dec_w, dec_b_2)

    return out


# speedup vs baseline: 1.1982x; 1.1982x over previous
import math

import jax
import jax.numpy as jnp
from jax.experimental import pallas as pl
from jax.experimental.pallas import tpu as pltpu

_LN_EPS = 1e-12
_SQRT_HALF = 1.0 / math.sqrt(2.0)


def _cdiv(a, b):
    return (a + b - 1) // b


def _transform_kernel(x_ref, w1_ref, b1_ref, gamma_ref, beta_ref, h_ref):
    x = x_ref[...].astype(jnp.bfloat16)
    w = w1_ref[...].astype(jnp.bfloat16)
    y = jax.lax.dot_general(x, w, (((1,), (1,)), ((), ())),
                            preferred_element_type=jnp.float32)
    y = y + b1_ref[...]
    y = y * 0.5 * (1.0 + jax.lax.erf(y * _SQRT_HALF))
    mean = jnp.mean(y, axis=-1, keepdims=True)
    centered = y - mean
    var = jnp.mean(centered * centered, axis=-1, keepdims=True)
    y = centered * jax.lax.rsqrt(var + _LN_EPS)
    y = y * gamma_ref[...] + beta_ref[...]
    h_ref[...] = y.astype(h_ref.dtype)


def _decoder_kernel(h_ref, wv_ref, bv_ref, out_ref):
    w = wv_ref[...].astype(jnp.bfloat16)
    logits = jax.lax.dot_general(h_ref[...], w, (((1,), (1,)), ((), ())),
                                 preferred_element_type=jnp.float32)
    out_ref[...] = logits + bv_ref[...]


def kernel(x, w1, b1, gamma, beta, dec_w, dec_b):
    B, S, H = x.shape
    V = dec_w.shape[0]
    M = B * S

    x2 = x.reshape(M, H)
    b1_2 = b1.reshape(1, H).astype(jnp.float32)
    gamma_2 = gamma.reshape(1, H).astype(jnp.float32)
    beta_2 = beta.reshape(1, H).astype(jnp.float32)
    dec_b_2 = dec_b.reshape(1, V).astype(jnp.float32)

    tm = min(512, M)
    h = pl.pallas_call(
        _transform_kernel,
        out_shape=jax.ShapeDtypeStruct((M, H), jnp.bfloat16),
        grid=(_cdiv(M, tm),),
        in_specs=[
            pl.BlockSpec((tm, H), lambda i: (i, 0)),
            pl.BlockSpec((H, H), lambda i: (0, 0)),
            pl.BlockSpec((1, H), lambda i: (0, 0)),
            pl.BlockSpec((1, H), lambda i: (0, 0)),
            pl.BlockSpec((1, H), lambda i: (0, 0)),
        ],
        out_specs=pl.BlockSpec((tm, H), lambda i: (i, 0)),
        compiler_params=pltpu.CompilerParams(
            dimension_semantics=("parallel",),
            vmem_limit_bytes=64 * 1024 * 1024,
        ),
        cost_estimate=pl.CostEstimate(
            flops=2 * M * H * H,
            transcendentals=M * H,
            bytes_accessed=4 * (M * H + H * H + 3 * H) + 2 * M * H,
        ),
    )(x2, w1, b1_2, gamma_2, beta_2)

    tv = 1280
    out = pl.pallas_call(
        _decoder_kernel,
        out_shape=jax.ShapeDtypeStruct((M, V), jnp.float32),
        grid=(_cdiv(V, tv),),
        in_specs=[
            pl.BlockSpec((M, H), lambda j: (0, 0)),
            pl.BlockSpec((tv, H), lambda j: (j, 0)),
            pl.BlockSpec((1, tv), lambda j: (0, j)),
        ],
        out_specs=pl.BlockSpec((M, tv), lambda j: (0, j)),
        compiler_params=pltpu.CompilerParams(
            dimension_semantics=("parallel",),
            vmem_limit_bytes=64 * 1024 * 1024,
        ),
        cost_estimate=pl.CostEstimate(
            flops=2 * M * H * V,
            transcendentals=0,
            bytes_accessed=2 * M * H + 4 * (H * V + V + M * V),
        ),
    )(h, dec_w, dec_b_2)

    return out.reshape(B, S, V)
